# baseline (speedup 1.0000x reference)
import jax
import jax.numpy as jnp
from jax import lax
from jax.experimental import pallas as pl
from jax.experimental.pallas import tpu as pltpu

N = 8192
TM = 512


def _dist_kernel(pred_ref, gtt_ref, row_ref, col_ref):
    i = pl.program_id(0)
    nsteps = pl.num_programs(0)
    p = pred_ref[...]
    g = gtt_ref[...]
    pb = p.astype(jnp.bfloat16)
    gneg = (-2.0 * g).astype(jnp.bfloat16)
    p2 = jnp.sum(p * p, axis=1, keepdims=True)
    g2 = jnp.sum(g * g, axis=0, keepdims=True)
    dot2 = jnp.dot(pb, gneg, preferred_element_type=jnp.float32)
    rmin = p2 + jnp.min(g2 + dot2, axis=1, keepdims=True)
    cpart = jnp.min(p2 + dot2, axis=0, keepdims=True)

    row_ref[pl.ds(i * TM, TM), :] = rmin

    @pl.when(i == 0)
    def _():
        col_ref[...] = cpart

    @pl.when(i != 0)
    def _():
        col_ref[...] = jnp.minimum(col_ref[...], cpart)

    @pl.when(i == nsteps - 1)
    def _():
        col_ref[...] = col_ref[...] + g2


def _finalize_kernel(row_ref, col_ref, acc_ref, comp_ref):
    rows = jnp.sqrt(jnp.maximum(row_ref[...], 0.0))
    cols = jnp.sqrt(jnp.maximum(col_ref[...], 0.0))

    comp = jnp.sum((cols < 0.05).astype(jnp.float32)) * (100.0 / N)
    comp_ref[...] = comp.reshape(1, 1)

    bits = lax.bitcast_convert_type(rows, jnp.int32)

    def kth_value(k):
        def body(_, carry):
            lo, hi = carry
            mid = lo + (hi - lo) // 2
            cnt = jnp.sum((bits <= mid).astype(jnp.int32))
            ge = cnt >= k + 1
            return (jnp.where(ge, lo, mid + 1), jnp.where(ge, mid, hi))

        lo, hi = lax.fori_loop(
            0, 31, body,
            (jnp.int32(0), jnp.int32(0x7F000000)))
        return jnp.min(jnp.where(bits >= hi, rows, jnp.float32(jnp.inf)))

    v1 = kth_value(7371)
    v2 = kth_value(7372)
    acc_ref[...] = (v1 + 0.9 * (v2 - v1)).reshape(1, 1)


def _pd_metrics(pred, gt, interpret=False):
    gtt = gt.T
    row_min2, col_min2 = pl.pallas_call(
        _dist_kernel,
        grid=(N // TM,),
        in_specs=[
            pl.BlockSpec((TM, 3), lambda i: (i, 0)),
            pl.BlockSpec((3, N), lambda i: (0, 0)),
        ],
        out_specs=[
            pl.BlockSpec((N, 1), lambda i: (0, 0)),
            pl.BlockSpec((1, N), lambda i: (0, 0)),
        ],
        out_shape=[
            jax.ShapeDtypeStruct((N, 1), jnp.float32),
            jax.ShapeDtypeStruct((1, N), jnp.float32),
        ],
        interpret=interpret,
    )(pred, gtt)

    rows = row_min2.reshape(64, 128)
    cols = col_min2.reshape(64, 128)
    acc, comp = pl.pallas_call(
        _finalize_kernel,
        out_shape=[
            jax.ShapeDtypeStruct((1, 1), jnp.float32),
            jax.ShapeDtypeStruct((1, 1), jnp.float32),
        ],
        interpret=interpret,
    )(rows, cols)
    return acc[0, 0], comp[0, 0]


def kernel(pred, gt):
    return _pd_metrics(pred, gt)


# speedup vs baseline: 1.1479x; 1.1479x over previous
import jax
import jax.numpy as jnp
from jax import lax
from jax.experimental import pallas as pl
from jax.experimental.pallas import tpu as pltpu

N = 8192
TM = 512


def _split3_bf16(x):
    a = x.astype(jnp.bfloat16)
    r1 = x - a.astype(jnp.float32)
    b = r1.astype(jnp.bfloat16)
    r2 = r1 - b.astype(jnp.float32)
    c = r2.astype(jnp.bfloat16)
    return a, b, c


def _pd_kernel(gt_ref, predt_ref, acc_ref, comp_ref, rhs_ref, colacc_ref,
               cnt_ref):
    i = pl.program_id(0)
    nsteps = pl.num_programs(0)

    @pl.when(i == 0)
    def _():
        predt = predt_ref[...]
        p2 = jnp.sum(predt * predt, axis=0, keepdims=True)
        pa, pb_, pc = _split3_bf16(p2)
        pneg = (-2.0 * predt).astype(jnp.bfloat16)
        ones = jnp.ones((3, N), jnp.bfloat16)
        rhs_ref[...] = jnp.concatenate([pneg, ones, pa, pb_, pc], axis=0)
        cnt_ref[0] = jnp.int32(0)

    g = gt_ref[...]
    g2 = jnp.sum(g * g, axis=1, keepdims=True)
    ga, gb_, gc = _split3_bf16(g2)
    lhs = jnp.concatenate(
        [g.astype(jnp.bfloat16), ga, gb_, gc,
         jnp.ones((TM, 3), jnp.bfloat16)], axis=1)

    d2 = jnp.dot(lhs, rhs_ref[...],
                 preferred_element_type=jnp.float32)

    rmin = jnp.min(d2, axis=1, keepdims=True)
    rdist = jnp.sqrt(jnp.maximum(rmin, 0.0))
    cnt_ref[0] += jnp.sum((rdist < 0.05).astype(jnp.int32))

    cmin = jnp.min(d2, axis=0, keepdims=True)

    @pl.when(i == 0)
    def _():
        colacc_ref[...] = cmin

    @pl.when(i != 0)
    def _():
        colacc_ref[...] = jnp.minimum(colacc_ref[...], cmin)

    @pl.when(i == nsteps - 1)
    def _():
        s = jnp.sqrt(jnp.maximum(colacc_ref[...], 0.0))
        bits = lax.bitcast_convert_type(s, jnp.int32)

        def kth_value(k):
            def body(_, carry):
                lo, hi = carry
                mid = lo + (hi - lo) // 2
                cnt = jnp.sum((bits <= mid).astype(jnp.int32))
                ge = cnt >= k + 1
                return (jnp.where(ge, lo, mid + 1), jnp.where(ge, mid, hi))

            lo, hi = lax.fori_loop(0, 31, body,
                                   (jnp.int32(0), jnp.int32(0x7F000000)))
            return jnp.min(jnp.where(bits >= hi, s, jnp.float32(jnp.inf)))

        v1 = kth_value(7371)
        v2 = kth_value(7372)
        acc_ref[...] = (v1 + 0.9 * (v2 - v1)).reshape(1, 1)
        comp_ref[...] = (cnt_ref[0].astype(jnp.float32)
                         * (100.0 / N)).reshape(1, 1)


def _pd_metrics(pred, gt, interpret=False):
    predt = pred.T
    acc, comp = pl.pallas_call(
        _pd_kernel,
        grid=(N // TM,),
        in_specs=[
            pl.BlockSpec((TM, 3), lambda i: (i, 0)),
            pl.BlockSpec((3, N), lambda i: (0, 0)),
        ],
        out_specs=[
            pl.BlockSpec((1, 1), lambda i: (0, 0)),
            pl.BlockSpec((1, 1), lambda i: (0, 0)),
        ],
        out_shape=[
            jax.ShapeDtypeStruct((1, 1), jnp.float32),
            jax.ShapeDtypeStruct((1, 1), jnp.float32),
        ],
        scratch_shapes=[
            pltpu.VMEM((9, N), jnp.bfloat16),
            pltpu.VMEM((1, N), jnp.float32),
            pltpu.SMEM((1,), jnp.int32),
        ],
        interpret=interpret,
    )(gt, predt)
    return acc[0, 0], comp[0, 0]


def kernel(pred, gt):
    return _pd_metrics(pred, gt)


# speedup vs baseline: 1.2074x; 1.0519x over previous
import jax
import jax.numpy as jnp
from jax import lax
from jax.experimental import pallas as pl
from jax.experimental.pallas import tpu as pltpu

N = 8192
TM = 1024


def _split3_bf16(x):
    a = x.astype(jnp.bfloat16)
    r1 = x - a.astype(jnp.float32)
    b = r1.astype(jnp.bfloat16)
    r2 = r1 - b.astype(jnp.float32)
    c = r2.astype(jnp.bfloat16)
    return a, b, c


def _pd_kernel(gt_ref, predt_ref, acc_ref, comp_ref, rhs_ref, colacc_ref,
               cnt_ref):
    i = pl.program_id(0)
    nsteps = pl.num_programs(0)

    @pl.when(i == 0)
    def _():
        predt = predt_ref[...]
        p2 = jnp.sum(predt * predt, axis=0, keepdims=True)
        pa, pb_, pc = _split3_bf16(p2)
        pneg = (-2.0 * predt).astype(jnp.bfloat16)
        ones = jnp.ones((3, N), jnp.bfloat16)
        rhs_ref[...] = jnp.concatenate([pneg, ones, pa, pb_, pc], axis=0)
        cnt_ref[0] = jnp.int32(0)

    g = gt_ref[...]
    g2 = jnp.sum(g * g, axis=1, keepdims=True)
    ga, gb_, gc = _split3_bf16(g2)
    lhs = jnp.concatenate(
        [g.astype(jnp.bfloat16), ga, gb_, gc,
         jnp.ones((TM, 3), jnp.bfloat16)], axis=1)

    d2 = jnp.dot(lhs, rhs_ref[...],
                 preferred_element_type=jnp.float32)

    rmin = jnp.min(d2, axis=1, keepdims=True)
    rdist = jnp.sqrt(jnp.maximum(rmin, 0.0))
    cnt_ref[0] += jnp.sum((rdist < 0.05).astype(jnp.int32))

    cmin = jnp.min(d2, axis=0, keepdims=True)

    @pl.when(i == 0)
    def _():
        colacc_ref[...] = cmin

    @pl.when(i != 0)
    def _():
        colacc_ref[...] = jnp.minimum(colacc_ref[...], cmin)

    @pl.when(i == nsteps - 1)
    def _():
        s = jnp.sqrt(jnp.maximum(colacc_ref[...], 0.0))
        bits = lax.bitcast_convert_type(s, jnp.int32)

        def kth_value(k):
            def body(_, carry):
                lo, hi = carry
                mid = lo + (hi - lo) // 2
                cnt = jnp.sum((bits <= mid).astype(jnp.int32))
                ge = cnt >= k + 1
                return (jnp.where(ge, lo, mid + 1), jnp.where(ge, mid, hi))

            lo, hi = lax.fori_loop(0, 31, body,
                                   (jnp.int32(0), jnp.int32(0x7F000000)))
            return jnp.min(jnp.where(bits >= hi, s, jnp.float32(jnp.inf)))

        v1 = kth_value(7371)
        v2 = kth_value(7372)
        acc_ref[...] = (v1 + 0.9 * (v2 - v1)).reshape(1, 1)
        comp_ref[...] = (cnt_ref[0].astype(jnp.float32)
                         * (100.0 / N)).reshape(1, 1)


def _pd_metrics(pred, gt, interpret=False):
    predt = pred.T
    acc, comp = pl.pallas_call(
        _pd_kernel,
        grid=(N // TM,),
        in_specs=[
            pl.BlockSpec((TM, 3), lambda i: (i, 0)),
            pl.BlockSpec((3, N), lambda i: (0, 0)),
        ],
        out_specs=[
            pl.BlockSpec((1, 1), lambda i: (0, 0)),
            pl.BlockSpec((1, 1), lambda i: (0, 0)),
        ],
        out_shape=[
            jax.ShapeDtypeStruct((1, 1), jnp.float32),
            jax.ShapeDtypeStruct((1, 1), jnp.float32),
        ],
        scratch_shapes=[
            pltpu.VMEM((9, N), jnp.bfloat16),
            pltpu.VMEM((1, N), jnp.float32),
            pltpu.SMEM((1,), jnp.int32),
        ],
        interpret=interpret,
    )(gt, predt)
    return acc[0, 0], comp[0, 0]


def kernel(pred, gt):
    return _pd_metrics(pred, gt)
